# baseline (speedup 1.0000x reference)
import jax
import jax.numpy as jnp
from jax import lax
from jax.experimental import pallas as pl
from jax.experimental.pallas import tpu as pltpu
from jax.experimental.pallas import tpu_sc as plsc

N_G = 10000
N_G_PAD = 10240
N_P = 1000
N_P_PAD = 1024
D = 128
OUT = 16
NC = 2
NS = 16
CHUNK = 128
NB = 8
CG = 80
CP = 32


def _seg_body(msg_gg, msg_gp, ggs, ggd, ggw, gps, gpd, gpw,
              tg_out, tp_out,
              acc_g, acc_p, sv, dv, wv, rows, sem):
  c = lax.axis_index("c")
  s = lax.axis_index("s")
  t = c * NS + s

  def zrow(i, _):
    for d8 in range(8):
      rows[i, pl.ds(d8 * 16, 16)] = jnp.zeros((16,), jnp.float32)
    return 0
  lax.fori_loop(0, CHUNK, zrow, 0)

  base_g = s * 624
  for i in range(4):
    pltpu.sync_copy(rows, acc_g.at[pl.ds(base_g + i * 128, 128)])
  pltpu.sync_copy(rows.at[pl.ds(0, 112)], acc_g.at[pl.ds(base_g + 512, 112)])
  pltpu.sync_copy(rows.at[pl.ds(0, 64)], acc_p.at[pl.ds(s * 64, 64)])
  @pl.when(s == 0)
  def _():
    pltpu.sync_copy(rows, acc_g.at[pl.ds(9984, 128)])
    pltpu.sync_copy(rows, acc_g.at[pl.ds(10112, 128)])
  plsc.subcore_barrier()

  def do_rel(n_blocks, srcs, dsts, ws, msg, acc):
    def block_body(bb, _):
      off = pl.multiple_of(bb * NB, NB)
      pltpu.sync_copy(srcs.at[t, pl.ds(off, NB)], sv)
      pltpu.sync_copy(dsts.at[t, pl.ds(off, NB)], dv)
      pltpu.sync_copy(ws.at[t, pl.ds(off, NB)], wv)

      def chunk_body(jj, _):
        pltpu.async_copy(msg.at[sv.at[jj]], rows, sem).wait()

        def scale(g, _):
          wvec = wv[jj, pl.ds(g * 16, 16)]
          for lane in range(16):
            sc = wvec[lane]
            k = g * 16 + lane
            for d8 in range(8):
              sl = pl.ds(d8 * 16, 16)
              rows[k, sl] = rows[k, sl] * sc
          return 0
        lax.fori_loop(0, CHUNK // 16, scale, 0)
        pltpu.sync_copy(rows, acc.at[dv.at[jj]], add=True)
        return 0
      lax.fori_loop(0, NB, chunk_body, 0)
      return 0
    lax.fori_loop(0, n_blocks, block_body, 0)

  do_rel(CG // NB, ggs, ggd, ggw, msg_gg, acc_g)
  do_rel(CP // NB, gps, gpd, gpw, msg_gp, acc_p)
  plsc.subcore_barrier()

  pltpu.sync_copy(acc_g.at[pl.ds(base_g, 624)], tg_out.at[c, pl.ds(base_g, 624)])
  pltpu.sync_copy(acc_p.at[pl.ds(s * 64, 64)], tp_out.at[c, pl.ds(s * 64, 64)])
  @pl.when(s == 0)
  def _():
    pltpu.sync_copy(acc_g.at[pl.ds(9984, 16)], tg_out.at[c, pl.ds(9984, 16)])


def _seg_kernel():
  return pl.kernel(
    _seg_body,
    out_type=(
        jax.ShapeDtypeStruct((NC, N_G, D), jnp.float32),
        jax.ShapeDtypeStruct((NC, N_P_PAD, D), jnp.float32),
    ),
    mesh=plsc.VectorSubcoreMesh(
        core_axis_name="c", subcore_axis_name="s", num_cores=NC,
        num_subcores=NS),
    scratch_types=(
        pltpu.VMEM_SHARED((N_G_PAD, D), jnp.float32),
        pltpu.VMEM_SHARED((N_P_PAD, D), jnp.float32),
        pltpu.VMEM((NB, CHUNK), jnp.int32),
        pltpu.VMEM((NB, CHUNK), jnp.int32),
        pltpu.VMEM((NB, CHUNK), jnp.float32),
        pltpu.VMEM((CHUNK, D), jnp.float32),
        pltpu.SemaphoreType.DMA,
    ),
  )


def _mm2_body(x_ref, w1_ref, b1_ref, w2_ref, b2_ref, o1_ref, o2_ref):
  x = x_ref[...]
  o1_ref[...] = jnp.dot(x, w1_ref[...],
                        preferred_element_type=jnp.float32) + b1_ref[...]
  o2_ref[...] = jnp.dot(x, w2_ref[...],
                        preferred_element_type=jnp.float32) + b2_ref[...]


def _mm2(h, w1, b1, w2, b2):
  blk = 2000
  grid = N_G // blk
  return pl.pallas_call(
      _mm2_body,
      grid=(grid,),
      in_specs=[
          pl.BlockSpec((blk, D), lambda i: (i, 0)),
          pl.BlockSpec((D, D), lambda i: (0, 0)),
          pl.BlockSpec((1, D), lambda i: (0, 0)),
          pl.BlockSpec((D, D), lambda i: (0, 0)),
          pl.BlockSpec((1, D), lambda i: (0, 0)),
      ],
      out_specs=[
          pl.BlockSpec((blk, D), lambda i: (i, 0)),
          pl.BlockSpec((blk, D), lambda i: (i, 0)),
      ],
      out_shape=[
          jax.ShapeDtypeStruct((N_G, D), jnp.float32),
          jax.ShapeDtypeStruct((N_G, D), jnp.float32),
      ],
  )(h, w1, b1.reshape(1, D), w2, b2.reshape(1, D))


def _finish_math(t, h, wa, ba, g, b, sk):
  z = jnp.dot(t, wa, preferred_element_type=jnp.float32) + ba
  a = jax.nn.sigmoid(sk)
  u = z * a + h * (1.0 - a)
  m = jnp.mean(u, axis=-1, keepdims=True)
  v = jnp.mean((u - m) ** 2, axis=-1, keepdims=True)
  return (u - m) * lax.rsqrt(v + 1e-5) * g + b


def _fin_body(sk_ref, t_ref, h_ref, wa_ref, ba_ref, g_ref, b_ref, o_ref):
  t = t_ref[0] + t_ref[1]
  o_ref[...] = _finish_math(t, h_ref[...], wa_ref[...], ba_ref[...],
                            g_ref[...], b_ref[...], sk_ref[0])


def _fin(tpart, h, wa, ba, g, b, sk, n, blk):
  grid = n // blk
  return pl.pallas_call(
      _fin_body,
      grid=(grid,),
      in_specs=[
          pl.BlockSpec(memory_space=pltpu.SMEM),
          pl.BlockSpec((NC, blk, D), lambda i: (0, i, 0)),
          pl.BlockSpec((blk, D), lambda i: (i, 0)),
          pl.BlockSpec((D, D), lambda i: (0, 0)),
          pl.BlockSpec((1, D), lambda i: (0, 0)),
          pl.BlockSpec((1, D), lambda i: (0, 0)),
          pl.BlockSpec((1, D), lambda i: (0, 0)),
      ],
      out_specs=pl.BlockSpec((blk, D), lambda i: (i, 0)),
      out_shape=jax.ShapeDtypeStruct((n, D), jnp.float32),
  )(jnp.reshape(sk, (1,)), tpart, h, wa, ba.reshape(1, D), g.reshape(1, D),
    b.reshape(1, D))


def _fin_head_body(sk_ref, t_ref, h_ref, wa_ref, ba_ref, g_ref, b_ref,
                   wo_ref, bo_ref, o_ref):
  t = t_ref[0] + t_ref[1]
  hp = _finish_math(t, h_ref[...], wa_ref[...], ba_ref[...],
                    g_ref[...], b_ref[...], sk_ref[0])
  o_ref[...] = jnp.dot(hp, wo_ref[...],
                       preferred_element_type=jnp.float32) + bo_ref[...]


def _fin_head(tpart, h, wa, ba, g, b, sk, wo, bo):
  return pl.pallas_call(
      _fin_head_body,
      grid=(1,),
      in_specs=[
          pl.BlockSpec(memory_space=pltpu.SMEM),
          pl.BlockSpec((NC, N_P, D), lambda i: (0, 0, 0)),
          pl.BlockSpec((N_P, D), lambda i: (0, 0)),
          pl.BlockSpec((D, D), lambda i: (0, 0)),
          pl.BlockSpec((1, D), lambda i: (0, 0)),
          pl.BlockSpec((1, D), lambda i: (0, 0)),
          pl.BlockSpec((1, D), lambda i: (0, 0)),
          pl.BlockSpec((D, OUT), lambda i: (0, 0)),
          pl.BlockSpec((1, OUT), lambda i: (0, 0)),
      ],
      out_specs=pl.BlockSpec((N_P, OUT), lambda i: (0, 0)),
      out_shape=jax.ShapeDtypeStruct((N_P, OUT), jnp.float32),
  )(jnp.reshape(sk, (1,)), tpart, h, wa, ba.reshape(1, D), g.reshape(1, D),
    b.reshape(1, D), wo, bo.reshape(1, OUT))


def _pad_edges(src, dst, w, n_chunks_tile, pad_row_base, pad_row_span):
  e = src.shape[0]
  tot = NC * NS * n_chunks_tile * CHUNK
  pad = tot - e
  shape = (NC * NS, n_chunks_tile, CHUNK)
  pad_dst = pad_row_base + (jnp.arange(pad, dtype=jnp.int32) % pad_row_span)
  src = jnp.pad(src, (0, pad)).reshape(shape)
  dst = jnp.concatenate([dst, pad_dst]).reshape(shape)
  w = jnp.pad(w, (0, pad)).reshape(shape)
  return src, dst, w


def kernel(nid_gene, nid_patient, gg_src, gg_dst, gp_src, gp_dst, w_gg, w_gp,
           emb_gene, emb_patient, Wr_gg, br_gg, Wr_gp, br_gp,
           Wa_g, ba_g, Wa_p, ba_p, ln_g_w, ln_g_b, ln_p_w, ln_p_b,
           skip, W_out, b_out):
  h_g = jnp.take(emb_gene, nid_gene, axis=0)
  h_p = jnp.take(emb_patient, nid_patient, axis=0)

  ggs, ggd, ggw = _pad_edges(gg_src, gg_dst, w_gg, CG, N_G, N_G_PAD - N_G)
  gps, gpd, gpw = _pad_edges(gp_src, gp_dst, w_gp, CP, N_P, N_P_PAD - N_P)

  seg = _seg_kernel()

  for l in range(2):
    msg_gg, msg_gp = _mm2(h_g, Wr_gg[l], br_gg[l], Wr_gp[l], br_gp[l])
    tg_part, tp_part = seg(msg_gg, msg_gp, ggs, ggd, ggw, gps, gpd, gpw)
    tp_part = tp_part[:, :N_P]
    if l == 0:
      h_g = _fin(tg_part, h_g, Wa_g[l], ba_g[l], ln_g_w[l], ln_g_b[l],
                 skip[l, 0], N_G, 2000)
      h_p = _fin(tp_part, h_p, Wa_p[l], ba_p[l], ln_p_w[l], ln_p_b[l],
                 skip[l, 1], N_P, N_P)
    else:
      logits = _fin_head(tp_part, h_p, Wa_p[l], ba_p[l], ln_p_w[l],
                         ln_p_b[l], skip[l, 1], W_out, b_out)
  return logits


# speedup vs baseline: 1.0611x; 1.0611x over previous
import jax
import jax.numpy as jnp
from jax import lax
from jax.experimental import pallas as pl
from jax.experimental.pallas import tpu as pltpu
from jax.experimental.pallas import tpu_sc as plsc

N_G = 10000
N_G_PAD = 10112
N_P = 1000
N_P_PAD = 1024
D = 128
OUT = 16
NC = 2
NS = 16
CHUNK = 128
NB = 8
CG = 80
CP = 32


def _seg_body(msg_gg, msg_gp, ggs, ggd, ggw, gps, gpd, gpw,
              tg_out, tp_out,
              acc_g, acc_p, sv, dv, wv, rows0, rows1, sem0, sem1):
  rows_bufs = (rows0, rows1)
  sems = (sem0, sem1)
  rows = rows0
  c = lax.axis_index("c")
  s = lax.axis_index("s")
  t = c * NS + s

  def zrow(i, _):
    for d8 in range(8):
      rows[i, pl.ds(d8 * 16, 16)] = jnp.zeros((16,), jnp.float32)
    return 0
  lax.fori_loop(0, CHUNK, zrow, 0)

  base_g = s * 624
  for i in range(4):
    pltpu.sync_copy(rows, acc_g.at[pl.ds(base_g + i * 128, 128)])
  pltpu.sync_copy(rows.at[pl.ds(0, 112)], acc_g.at[pl.ds(base_g + 512, 112)])
  pltpu.sync_copy(rows.at[pl.ds(0, 64)], acc_p.at[pl.ds(s * 64, 64)])
  @pl.when(s == 0)
  def _():
    pltpu.sync_copy(rows, acc_g.at[pl.ds(9984, 128)])
  plsc.subcore_barrier()

  def do_rel(n_blocks, srcs, dsts, ws, msg, acc):
    def scale(buf, jj):
      def scale_g(g, _):
        wvec = wv[jj, pl.ds(g * 16, 16)]
        for lane in range(16):
          sc = wvec[lane]
          k = g * 16 + lane
          for d8 in range(8):
            sl = pl.ds(d8 * 16, 16)
            buf[k, sl] = buf[k, sl] * sc
        return 0
      lax.fori_loop(0, CHUNK // 16, scale_g, 0)

    def block_body(bb, _):
      off = pl.multiple_of(bb * NB, NB)
      pltpu.sync_copy(srcs.at[t, pl.ds(off, NB)], sv)
      pltpu.sync_copy(dsts.at[t, pl.ds(off, NB)], dv)
      pltpu.sync_copy(ws.at[t, pl.ds(off, NB)], wv)

      pltpu.async_copy(msg.at[sv.at[0]], rows_bufs[0], sems[0])
      for jj in range(NB):
        cur, csem = rows_bufs[jj % 2], sems[jj % 2]
        pltpu.make_async_copy(msg.at[sv.at[jj]], cur, csem).wait()
        if jj + 1 < NB:
          pltpu.async_copy(msg.at[sv.at[jj + 1]], rows_bufs[(jj + 1) % 2],
                           sems[(jj + 1) % 2])
        scale(cur, jj)
        pltpu.sync_copy(cur, acc.at[dv.at[jj]], add=True)
      return 0
    lax.fori_loop(0, n_blocks, block_body, 0)

  do_rel(CG // NB, ggs, ggd, ggw, msg_gg, acc_g)
  do_rel(CP // NB, gps, gpd, gpw, msg_gp, acc_p)
  plsc.subcore_barrier()

  pltpu.sync_copy(acc_g.at[pl.ds(base_g, 624)], tg_out.at[c, pl.ds(base_g, 624)])
  pltpu.sync_copy(acc_p.at[pl.ds(s * 64, 64)], tp_out.at[c, pl.ds(s * 64, 64)])
  @pl.when(s == 0)
  def _():
    pltpu.sync_copy(acc_g.at[pl.ds(9984, 16)], tg_out.at[c, pl.ds(9984, 16)])


def _seg_kernel():
  return pl.kernel(
    _seg_body,
    out_type=(
        jax.ShapeDtypeStruct((NC, N_G, D), jnp.float32),
        jax.ShapeDtypeStruct((NC, N_P_PAD, D), jnp.float32),
    ),
    mesh=plsc.VectorSubcoreMesh(
        core_axis_name="c", subcore_axis_name="s", num_cores=NC,
        num_subcores=NS),
    scratch_types=(
        pltpu.VMEM_SHARED((N_G_PAD, D), jnp.float32),
        pltpu.VMEM_SHARED((N_P_PAD, D), jnp.float32),
        pltpu.VMEM((NB, CHUNK), jnp.int32),
        pltpu.VMEM((NB, CHUNK), jnp.int32),
        pltpu.VMEM((NB, CHUNK), jnp.float32),
        pltpu.VMEM((CHUNK, D), jnp.float32),
        pltpu.VMEM((CHUNK, D), jnp.float32),
        pltpu.SemaphoreType.DMA,
        pltpu.SemaphoreType.DMA,
    ),
  )


def _mm2_body(x_ref, w1_ref, b1_ref, w2_ref, b2_ref, o1_ref, o2_ref):
  x = x_ref[...]
  o1_ref[...] = jnp.dot(x, w1_ref[...],
                        preferred_element_type=jnp.float32) + b1_ref[...]
  o2_ref[...] = jnp.dot(x, w2_ref[...],
                        preferred_element_type=jnp.float32) + b2_ref[...]


def _mm2(h, w1, b1, w2, b2):
  blk = 2000
  grid = N_G // blk
  return pl.pallas_call(
      _mm2_body,
      grid=(grid,),
      in_specs=[
          pl.BlockSpec((blk, D), lambda i: (i, 0)),
          pl.BlockSpec((D, D), lambda i: (0, 0)),
          pl.BlockSpec((1, D), lambda i: (0, 0)),
          pl.BlockSpec((D, D), lambda i: (0, 0)),
          pl.BlockSpec((1, D), lambda i: (0, 0)),
      ],
      out_specs=[
          pl.BlockSpec((blk, D), lambda i: (i, 0)),
          pl.BlockSpec((blk, D), lambda i: (i, 0)),
      ],
      out_shape=[
          jax.ShapeDtypeStruct((N_G, D), jnp.float32),
          jax.ShapeDtypeStruct((N_G, D), jnp.float32),
      ],
  )(h, w1, b1.reshape(1, D), w2, b2.reshape(1, D))


def _finish_math(t, h, wa, ba, g, b, sk):
  z = jnp.dot(t, wa, preferred_element_type=jnp.float32) + ba
  a = jax.nn.sigmoid(sk)
  u = z * a + h * (1.0 - a)
  m = jnp.mean(u, axis=-1, keepdims=True)
  v = jnp.mean((u - m) ** 2, axis=-1, keepdims=True)
  return (u - m) * lax.rsqrt(v + 1e-5) * g + b


def _fin_body(sk_ref, t_ref, h_ref, wa_ref, ba_ref, g_ref, b_ref, o_ref):
  t = t_ref[0] + t_ref[1]
  o_ref[...] = _finish_math(t, h_ref[...], wa_ref[...], ba_ref[...],
                            g_ref[...], b_ref[...], sk_ref[0])


def _fin(tpart, h, wa, ba, g, b, sk, n, blk):
  grid = n // blk
  return pl.pallas_call(
      _fin_body,
      grid=(grid,),
      in_specs=[
          pl.BlockSpec(memory_space=pltpu.SMEM),
          pl.BlockSpec((NC, blk, D), lambda i: (0, i, 0)),
          pl.BlockSpec((blk, D), lambda i: (i, 0)),
          pl.BlockSpec((D, D), lambda i: (0, 0)),
          pl.BlockSpec((1, D), lambda i: (0, 0)),
          pl.BlockSpec((1, D), lambda i: (0, 0)),
          pl.BlockSpec((1, D), lambda i: (0, 0)),
      ],
      out_specs=pl.BlockSpec((blk, D), lambda i: (i, 0)),
      out_shape=jax.ShapeDtypeStruct((n, D), jnp.float32),
  )(jnp.reshape(sk, (1,)), tpart, h, wa, ba.reshape(1, D), g.reshape(1, D),
    b.reshape(1, D))


def _fin_head_body(sk_ref, t_ref, h_ref, wa_ref, ba_ref, g_ref, b_ref,
                   wo_ref, bo_ref, o_ref):
  t = t_ref[0] + t_ref[1]
  hp = _finish_math(t, h_ref[...], wa_ref[...], ba_ref[...],
                    g_ref[...], b_ref[...], sk_ref[0])
  o_ref[...] = jnp.dot(hp, wo_ref[...],
                       preferred_element_type=jnp.float32) + bo_ref[...]


def _fin_head(tpart, h, wa, ba, g, b, sk, wo, bo):
  return pl.pallas_call(
      _fin_head_body,
      grid=(1,),
      in_specs=[
          pl.BlockSpec(memory_space=pltpu.SMEM),
          pl.BlockSpec((NC, N_P, D), lambda i: (0, 0, 0)),
          pl.BlockSpec((N_P, D), lambda i: (0, 0)),
          pl.BlockSpec((D, D), lambda i: (0, 0)),
          pl.BlockSpec((1, D), lambda i: (0, 0)),
          pl.BlockSpec((1, D), lambda i: (0, 0)),
          pl.BlockSpec((1, D), lambda i: (0, 0)),
          pl.BlockSpec((D, OUT), lambda i: (0, 0)),
          pl.BlockSpec((1, OUT), lambda i: (0, 0)),
      ],
      out_specs=pl.BlockSpec((N_P, OUT), lambda i: (0, 0)),
      out_shape=jax.ShapeDtypeStruct((N_P, OUT), jnp.float32),
  )(jnp.reshape(sk, (1,)), tpart, h, wa, ba.reshape(1, D), g.reshape(1, D),
    b.reshape(1, D), wo, bo.reshape(1, OUT))


def _pad_edges(src, dst, w, n_chunks_tile, pad_row_base, pad_row_span):
  e = src.shape[0]
  tot = NC * NS * n_chunks_tile * CHUNK
  pad = tot - e
  shape = (NC * NS, n_chunks_tile, CHUNK)
  pad_dst = pad_row_base + (jnp.arange(pad, dtype=jnp.int32) % pad_row_span)
  src = jnp.pad(src, (0, pad)).reshape(shape)
  dst = jnp.concatenate([dst, pad_dst]).reshape(shape)
  w = jnp.pad(w, (0, pad)).reshape(shape)
  return src, dst, w


def kernel(nid_gene, nid_patient, gg_src, gg_dst, gp_src, gp_dst, w_gg, w_gp,
           emb_gene, emb_patient, Wr_gg, br_gg, Wr_gp, br_gp,
           Wa_g, ba_g, Wa_p, ba_p, ln_g_w, ln_g_b, ln_p_w, ln_p_b,
           skip, W_out, b_out):
  h_g = jnp.take(emb_gene, nid_gene, axis=0)
  h_p = jnp.take(emb_patient, nid_patient, axis=0)

  ggs, ggd, ggw = _pad_edges(gg_src, gg_dst, w_gg, CG, N_G, N_G_PAD - N_G)
  gps, gpd, gpw = _pad_edges(gp_src, gp_dst, w_gp, CP, N_P, N_P_PAD - N_P)

  seg = _seg_kernel()

  for l in range(2):
    msg_gg, msg_gp = _mm2(h_g, Wr_gg[l], br_gg[l], Wr_gp[l], br_gp[l])
    tg_part, tp_part = seg(msg_gg, msg_gp, ggs, ggd, ggw, gps, gpd, gpw)
    tp_part = tp_part[:, :N_P]
    if l == 0:
      h_g = _fin(tg_part, h_g, Wa_g[l], ba_g[l], ln_g_w[l], ln_g_b[l],
                 skip[l, 0], N_G, 2000)
      h_p = _fin(tp_part, h_p, Wa_p[l], ba_p[l], ln_p_w[l], ln_p_b[l],
                 skip[l, 1], N_P, N_P)
    else:
      logits = _fin_head(tp_part, h_p, Wa_p[l], ba_p[l], ln_p_w[l],
                         ln_p_b[l], skip[l, 1], W_out, b_out)
  return logits
